# baseline (speedup 1.0000x reference)
import functools

import jax
import jax.numpy as jnp
from jax import lax
from jax.experimental import pallas as pl
from jax.experimental.pallas import tpu as pltpu
from jax.experimental.pallas import tpu_sc as plsc

N = 10000
E = 320000
C = 128
H = 4
DH = 32
L = 2

NSUB = 16
EW = E // NSUB
K = 80
NCH = EW // K
RW = 624
RREM = N - RW * NSUB
RB = 1000
CE = C + 16

_f32 = jnp.float32


def _proj_body(xc, xg, Wcg, bcg, Wgc, bgc, Ms0, Ms1, Md, bs0, bs1, bd,
               hse_cg, hse_gc, sdst):
    xcb = xc[...]
    xgb = xg[...]
    hse_cg[:, 0:C] = jnp.dot(xcb, Wcg[...], preferred_element_type=_f32) + bcg[...]
    hse_cg[:, C:CE] = (jnp.dot(xcb, Ms0[...], preferred_element_type=_f32)
                       + bs0[...])
    hse_gc[:, 0:C] = jnp.dot(xgb, Wgc[...], preferred_element_type=_f32) + bgc[...]
    hse_gc[:, C:CE] = (jnp.dot(xgb, Ms1[...], preferred_element_type=_f32)
                       + bs1[...])
    sdst[...] = (jnp.dot(xgb, Md[...][0:C, :], preferred_element_type=_f32)
                 + jnp.dot(xcb, Md[...][C:2 * C, :], preferred_element_type=_f32)
                 + bd[...])


def _proj_call(xc, xg, Wcg, bcg, Wgc, bgc, Ms0, Ms1, Md, bs0, bs1, bd):
    row = lambda i: (i, 0)
    full = lambda i: (0, 0)
    return pl.pallas_call(
        _proj_body,
        grid=(N // RB,),
        in_specs=[
            pl.BlockSpec((RB, C), row), pl.BlockSpec((RB, C), row),
            pl.BlockSpec((C, C), full), pl.BlockSpec((1, C), full),
            pl.BlockSpec((C, C), full), pl.BlockSpec((1, C), full),
            pl.BlockSpec((C, 16), full), pl.BlockSpec((C, 16), full),
            pl.BlockSpec((2 * C, 16), full),
            pl.BlockSpec((1, 16), full), pl.BlockSpec((1, 16), full),
            pl.BlockSpec((1, 16), full),
        ],
        out_specs=[pl.BlockSpec((RB, CE), row), pl.BlockSpec((RB, CE), row),
                   pl.BlockSpec((RB, 16), row)],
        out_shape=[jax.ShapeDtypeStruct((N, CE), _f32),
                   jax.ShapeDtypeStruct((N, CE), _f32),
                   jax.ShapeDtypeStruct((N, 16), _f32)],
    )(xc, xg, Wcg, bcg, Wgc, bgc, Ms0, Ms1, Md, bs0, bs1, bd)


def _out_body(aggg, aggc, Wg, bg, Wc, bc, xg, xc, Ex, yg, yc):
    ex = Ex[...]
    eg = aggg[...]
    sg = jnp.dot(1.0 / (eg[:, C:CE] + 1e-16), ex, preferred_element_type=_f32)
    ag = eg[:, 0:C] * sg
    yg[...] = jnp.maximum(
        jnp.dot(ag, Wg[...], preferred_element_type=_f32) + bg[...], 0.0) + xg[...]
    ec = aggc[...]
    sc = jnp.dot(1.0 / (ec[:, C:CE] + 1e-16), ex, preferred_element_type=_f32)
    ac = ec[:, 0:C] * sc
    yc[...] = jnp.maximum(
        jnp.dot(ac, Wc[...], preferred_element_type=_f32) + bc[...], 0.0) + xc[...]


def _out_call(aggg, aggc, Wg, bg, Wc, bc, xg, xc, Ex):
    row = lambda i: (i, 0)
    full = lambda i: (0, 0)
    return pl.pallas_call(
        _out_body,
        grid=(N // RB,),
        in_specs=[
            pl.BlockSpec((RB, CE), row), pl.BlockSpec((RB, CE), row),
            pl.BlockSpec((C, C), full), pl.BlockSpec((1, C), full),
            pl.BlockSpec((C, C), full), pl.BlockSpec((1, C), full),
            pl.BlockSpec((RB, C), row), pl.BlockSpec((RB, C), row),
            pl.BlockSpec((16, C), full),
        ],
        out_specs=[pl.BlockSpec((RB, C), row), pl.BlockSpec((RB, C), row)],
        out_shape=[jax.ShapeDtypeStruct((N, C), _f32),
                   jax.ShapeDtypeStruct((N, C), _f32)],
    )(aggg, aggc, Wg, bg, Wc, bc, xg, xc, Ex)


def _sc_edge(hse_cg, hse_gc, sdst, src_cg, dst_cg, src_gc, dst_gc):
    mesh = plsc.VectorSubcoreMesh(core_axis_name="c", subcore_axis_name="s")
    out_type = [jax.ShapeDtypeStruct((N, CE), _f32),
                jax.ShapeDtypeStruct((N, CE), _f32)]
    NB = 3
    scratch = (
        [pltpu.VMEM((K, CE), _f32)] * NB
        + [pltpu.VMEM((K, 16), _f32)] * NB
        + [pltpu.VMEM((K,), jnp.int32)] * NB
        + [pltpu.VMEM((K,), jnp.int32)] * NB
        + [pltpu.VMEM((K // 16, 16), jnp.int32)] * NB
        + [pltpu.VMEM((4 * K,), _f32)]
        + [pltpu.VMEM_SHARED((N, CE), _f32)]
        + [pltpu.SemaphoreType.DMA] * (3 * NB)
    )

    @functools.partial(
        pl.kernel, mesh=mesh, out_type=out_type, scratch_types=scratch,
        compiler_params=pltpu.CompilerParams(needs_layout_passes=False,
                                             use_tc_tiling_on_sc=False))
    def k(hscg, hsgc, stb, scg, dcg, sgc, dgc,
          aggg, aggc,
          r0_v, r1_v, r2_v, t0_v, t1_v, t2_v,
          s0_v, s1_v, s2_v, d0_v, d1_v, d2_v, i0_v, i1_v, i2_v,
          exb_v, agg_s,
          gs0, gs1, gs2, is0, is1, is2, ss0, ss1, ss2):
        rows = [r0_v, r1_v, r2_v]
        sdst = [t0_v, t1_v, t2_v]
        srcv = [s0_v, s1_v, s2_v]
        dstv = [d0_v, d1_v, d2_v]
        sciv = [i0_v, i1_v, i2_v]
        gsem = [gs0, gs1, gs2]
        isem = [is0, is1, is2]
        ssem = [ss0, ss1, ss2]
        rows_v = rows[0]
        cid = lax.axis_index("c")
        sid = lax.axis_index("s")

        lane = lax.iota(jnp.int32, 16)
        den_off = jnp.where(lane < 4, lane * K, 0)
        den_msk = jnp.where(lane < 4, 1.0, 0.0).astype(_f32)

        def run(hs, roff, srcE, dstE, aggo):
            def zrow(i, _):
                rows_v[i // (CE // 16), pl.ds((i % (CE // 16)) * 16, 16)] = (
                    jnp.zeros((16,), _f32))
                return 0
            lax.fori_loop(0, K * (CE // 16), zrow, 0)

            r0 = sid * RW
            def zcp(j, _):
                pltpu.sync_copy(rows_v, agg_s.at[pl.ds(r0 + j * K, K)])
                return 0
            lax.fori_loop(0, RW // K, zcp, 0)
            rem = RW - (RW // K) * K
            if rem:
                pltpu.sync_copy(rows_v.at[pl.ds(0, rem)],
                                agg_s.at[pl.ds(r0 + RW - rem, rem)])

            @pl.when(sid == NSUB - 1)
            def _():
                pltpu.sync_copy(rows_v.at[pl.ds(0, RREM)],
                                agg_s.at[pl.ds(RW * NSUB, RREM)])
            plsc.subcore_barrier()

            def compute(rv, tv, dv, cv):
                def lgrp(g, _):
                    e0 = g * 16
                    ids = jnp.full((16,), e0, jnp.int32) + lane
                    for h in range(H):
                        av = plsc.load_gather(
                            rv, [ids, jnp.full((16,), C + h, jnp.int32)])
                        bv = plsc.load_gather(
                            tv, [ids, jnp.full((16,), roff + h, jnp.int32)])
                        lv = av + bv
                        lv = jnp.where(lv >= 0.0, lv, 0.2 * lv)
                        exb_v[pl.ds(h * K + e0, 16)] = jnp.exp(lv)
                    return 0
                lax.fori_loop(0, K // 16, lgrp, 0)
                for t in range(K // 16):
                    cv[t, :] = dv[pl.ds(t * 16, 16)]

                def escale(eb, _):
                    e0 = eb * 4
                    for j in range(4):
                        e = e0 + j
                        esp = jnp.full((16,), e, jnp.int32)
                        dvv = plsc.load_gather(exb_v, [esp + den_off]) * den_msk
                        rv[e, pl.ds(C, 16)] = dvv
                        for h in range(H):
                            sp = plsc.load_gather(exb_v, [esp + h * K])
                            for q in range(2):
                                o = h * DH + q * 16
                                rv[e, pl.ds(o, 16)] = rv[e, pl.ds(o, 16)] * sp
                    return 0
                lax.fori_loop(0, K // 4, escale, 0)

            def do_chunk(ch, b):
                bn = (b + 1) % NB
                bn2 = (b + 2) % NB

                @pl.when((ch >= 1) & (ch + 1 < NCH))
                def _():
                    pltpu.make_async_copy(
                        srcE.at[pl.ds(0, K)], srcv[bn], isem[bn]).wait()
                    pltpu.make_async_copy(
                        dstE.at[pl.ds(0, K)], dstv[bn], isem[bn]).wait()

                @pl.when((ch >= 2) & (ch + 1 < NCH))
                def _():
                    for q in range(K // 16):
                        pltpu.make_async_copy(
                            rows[bn].at[pl.ds(q * 16, 16)],
                            agg_s.at[sciv[bn].at[q]], ssem[bn]).wait()

                @pl.when(ch + 1 < NCH)
                def _():
                    for q in range(K // 16):
                        sl = pl.ds(q * 16, 16)
                        pltpu.async_copy(hs.at[srcv[bn].at[sl]],
                                         rows[bn].at[sl], gsem[bn])
                        pltpu.async_copy(stb.at[dstv[bn].at[sl]],
                                         sdst[bn].at[sl], gsem[bn])

                @pl.when(ch + 2 < NCH)
                def _():
                    base2 = sid * EW + (ch + 2) * K
                    pltpu.async_copy(srcE.at[pl.ds(base2, K)], srcv[bn2],
                                     isem[bn2])
                    pltpu.async_copy(dstE.at[pl.ds(base2, K)], dstv[bn2],
                                     isem[bn2])

                for q in range(K // 16):
                    sl = pl.ds(q * 16, 16)
                    pltpu.make_async_copy(hs.at[srcv[b].at[sl]],
                                          rows[b].at[sl], gsem[b]).wait()
                    pltpu.make_async_copy(stb.at[dstv[b].at[sl]],
                                          sdst[b].at[sl], gsem[b]).wait()

                compute(rows[b], sdst[b], dstv[b], sciv[b])

                for q in range(K // 16):
                    pltpu.async_copy(rows[b].at[pl.ds(q * 16, 16)],
                                     agg_s.at[sciv[b].at[q]], ssem[b],
                                     add=True)

            base0 = sid * EW
            pltpu.sync_copy(srcE.at[pl.ds(base0, K)], srcv[0])
            pltpu.sync_copy(dstE.at[pl.ds(base0, K)], dstv[0])
            pltpu.sync_copy(srcE.at[pl.ds(base0 + K, K)], srcv[1])
            pltpu.sync_copy(dstE.at[pl.ds(base0 + K, K)], dstv[1])
            for q in range(K // 16):
                sl = pl.ds(q * 16, 16)
                pltpu.async_copy(hs.at[srcv[0].at[sl]], rows[0].at[sl],
                                 gsem[0])
                pltpu.async_copy(stb.at[dstv[0].at[sl]], sdst[0].at[sl],
                                 gsem[0])

            def loop_body(ch, _):
                r = lax.rem(ch, NB)
                for b in range(NB):
                    @pl.when(r == b)
                    def _(b=b):
                        do_chunk(ch, b)
                return 0
            lax.fori_loop(0, NCH, loop_body, 0)

            for j in (NCH - 3, NCH - 2, NCH - 1):
                bj = j % NB
                for q in range(K // 16):
                    pltpu.make_async_copy(
                        rows[bj].at[pl.ds(q * 16, 16)],
                        agg_s.at[sciv[bj].at[q]], ssem[bj]).wait()
            plsc.subcore_barrier()

            pltpu.sync_copy(agg_s.at[pl.ds(r0, RW)], aggo.at[pl.ds(r0, RW)])

            @pl.when(sid == NSUB - 1)
            def _():
                pltpu.sync_copy(agg_s.at[pl.ds(RW * NSUB, RREM)],
                                aggo.at[pl.ds(RW * NSUB, RREM)])

        @pl.when(cid == 0)
        def _():
            run(hscg, 0, scg, dcg, aggg)

        @pl.when(cid == 1)
        def _():
            run(hsgc, 8, sgc, dgc, aggc)

    return k(hse_cg, hse_gc, sdst, src_cg, dst_cg, src_gc, dst_gc)


def kernel(x_chemical, x_gene, edge_index_cg, edge_index_gc,
           Wsrc, bsrc, Wdst, bdst, attn, Wout, bout):
    xc, xg = x_chemical, x_gene
    src_cg, dst_cg = edge_index_cg[0], edge_index_cg[1]
    src_gc, dst_gc = edge_index_gc[0], edge_index_gc[1]

    eye4 = jnp.eye(H, dtype=_f32)
    Ex = jnp.concatenate(
        [jnp.repeat(eye4, DH, axis=1), jnp.zeros((12, C), _f32)], axis=0)
    z12 = jnp.zeros((C, 12), _f32)
    z4 = jnp.zeros((C, 4), _f32)
    zb4 = jnp.zeros((4,), _f32)
    zb12 = jnp.zeros((12,), _f32)

    for l in range(L):
        A0 = (attn[l, 0][:, :, None] * eye4[:, None, :]).reshape(C, H)
        A1 = (attn[l, 1][:, :, None] * eye4[:, None, :]).reshape(C, H)
        Wts0, bts0 = Wsrc[l, 0] @ A0, bsrc[l, 0] @ A0
        Wtd0, btd0 = Wdst[l, 0] @ A0, bdst[l, 0] @ A0
        Wts1, bts1 = Wsrc[l, 1] @ A1, bsrc[l, 1] @ A1
        Wtd1, btd1 = Wdst[l, 1] @ A1, bdst[l, 1] @ A1
        Ms0 = jnp.concatenate([Wts0, z12], axis=1)
        Ms1 = jnp.concatenate([Wts1, z12], axis=1)
        Md = jnp.concatenate([
            jnp.concatenate([Wtd0, z12], axis=1),
            jnp.concatenate([z4, z4, Wtd1, z4], axis=1),
        ], axis=0)
        bs0 = jnp.concatenate([bts0, zb12])[None]
        bs1 = jnp.concatenate([bts1, zb12])[None]
        bd = jnp.concatenate([btd0, zb4, btd1, zb4])[None]

        hse_cg, hse_gc, sdst = _proj_call(
            xc, xg, Wsrc[l, 0], bsrc[l, 0][None], Wsrc[l, 1], bsrc[l, 1][None],
            Ms0, Ms1, Md, bs0, bs1, bd)

        aggg, aggc = _sc_edge(
            hse_cg, hse_gc, sdst, src_cg, dst_cg, src_gc, dst_gc)

        xg, xc = _out_call(aggg, aggc,
                           Wout[l, 1], bout[l, 1][None],
                           Wout[l, 0], bout[l, 0][None], xg, xc, Ex)

    return jnp.concatenate([xc, xg], axis=0)


# speedup vs baseline: 2.2089x; 2.2089x over previous
import functools

import jax
import jax.numpy as jnp
from jax import lax
from jax.experimental import pallas as pl
from jax.experimental.pallas import tpu as pltpu
from jax.experimental.pallas import tpu_sc as plsc

N = 10000
E = 320000
C = 128
H = 4
DH = 32
L = 2

NSUB = 16
EW = E // NSUB
K = 80
NCH = EW // K
RW = 624
RREM = N - RW * NSUB
RB = 1000
CE = C + 16

_f32 = jnp.float32


def _proj_body(xc, xg, Wcg, bcg, Wgc, bgc, Ms0, Ms1, Md, bs0, bs1, bd,
               hse_cg, hse_gc, sdst):
    xcb = xc[...]
    xgb = xg[...]
    hse_cg[:, 0:C] = jnp.dot(xcb, Wcg[...], preferred_element_type=_f32) + bcg[...]
    hse_cg[:, C:CE] = (jnp.dot(xcb, Ms0[...], preferred_element_type=_f32)
                       + bs0[...])
    hse_gc[:, 0:C] = jnp.dot(xgb, Wgc[...], preferred_element_type=_f32) + bgc[...]
    hse_gc[:, C:CE] = (jnp.dot(xgb, Ms1[...], preferred_element_type=_f32)
                       + bs1[...])
    sdst[...] = (jnp.dot(xgb, Md[...][0:C, :], preferred_element_type=_f32)
                 + jnp.dot(xcb, Md[...][C:2 * C, :], preferred_element_type=_f32)
                 + bd[...])


def _proj_call(xc, xg, Wcg, bcg, Wgc, bgc, Ms0, Ms1, Md, bs0, bs1, bd):
    row = lambda i: (i, 0)
    full = lambda i: (0, 0)
    return pl.pallas_call(
        _proj_body,
        grid=(N // RB,),
        in_specs=[
            pl.BlockSpec((RB, C), row), pl.BlockSpec((RB, C), row),
            pl.BlockSpec((C, C), full), pl.BlockSpec((1, C), full),
            pl.BlockSpec((C, C), full), pl.BlockSpec((1, C), full),
            pl.BlockSpec((C, 16), full), pl.BlockSpec((C, 16), full),
            pl.BlockSpec((2 * C, 16), full),
            pl.BlockSpec((1, 16), full), pl.BlockSpec((1, 16), full),
            pl.BlockSpec((1, 16), full),
        ],
        out_specs=[pl.BlockSpec((RB, CE), row), pl.BlockSpec((RB, CE), row),
                   pl.BlockSpec((RB, 16), row)],
        out_shape=[jax.ShapeDtypeStruct((N, CE), _f32),
                   jax.ShapeDtypeStruct((N, CE), _f32),
                   jax.ShapeDtypeStruct((N, 16), _f32)],
    )(xc, xg, Wcg, bcg, Wgc, bgc, Ms0, Ms1, Md, bs0, bs1, bd)


def _out_body(aggg, aggc, Wg, bg, Wc, bc, xg, xc, Ex, yg, yc):
    ex = Ex[...]
    eg = aggg[...]
    sg = jnp.dot(1.0 / (eg[:, C:CE] + 1e-16), ex, preferred_element_type=_f32)
    ag = eg[:, 0:C] * sg
    yg[...] = jnp.maximum(
        jnp.dot(ag, Wg[...], preferred_element_type=_f32) + bg[...], 0.0) + xg[...]
    ec = aggc[...]
    sc = jnp.dot(1.0 / (ec[:, C:CE] + 1e-16), ex, preferred_element_type=_f32)
    ac = ec[:, 0:C] * sc
    yc[...] = jnp.maximum(
        jnp.dot(ac, Wc[...], preferred_element_type=_f32) + bc[...], 0.0) + xc[...]


def _out_call(aggg, aggc, Wg, bg, Wc, bc, xg, xc, Ex):
    row = lambda i: (i, 0)
    full = lambda i: (0, 0)
    return pl.pallas_call(
        _out_body,
        grid=(N // RB,),
        in_specs=[
            pl.BlockSpec((RB, CE), row), pl.BlockSpec((RB, CE), row),
            pl.BlockSpec((C, C), full), pl.BlockSpec((1, C), full),
            pl.BlockSpec((C, C), full), pl.BlockSpec((1, C), full),
            pl.BlockSpec((RB, C), row), pl.BlockSpec((RB, C), row),
            pl.BlockSpec((16, C), full),
        ],
        out_specs=[pl.BlockSpec((RB, C), row), pl.BlockSpec((RB, C), row)],
        out_shape=[jax.ShapeDtypeStruct((N, C), _f32),
                   jax.ShapeDtypeStruct((N, C), _f32)],
    )(aggg, aggc, Wg, bg, Wc, bc, xg, xc, Ex)


def _sc_edge(hse_cg, hse_gc, sdst, src_cg, dst_cg, src_gc, dst_gc):
    mesh = plsc.VectorSubcoreMesh(core_axis_name="c", subcore_axis_name="s")
    out_type = [jax.ShapeDtypeStruct((N, CE), _f32),
                jax.ShapeDtypeStruct((N, CE), _f32)]
    NB = 3
    scratch = (
        [pltpu.VMEM((K, CE), _f32)] * NB
        + [pltpu.VMEM((K, 16), _f32)] * NB
        + [pltpu.VMEM((K,), jnp.int32)] * NB
        + [pltpu.VMEM((K,), jnp.int32)] * NB
        + [pltpu.VMEM((K // 16, 16), jnp.int32)] * NB
        + [pltpu.VMEM((4 * K,), _f32)]
        + [pltpu.VMEM_SHARED((N, CE), _f32)]
        + [pltpu.SemaphoreType.DMA] * (3 * NB)
    )

    @functools.partial(
        pl.kernel, mesh=mesh, out_type=out_type, scratch_types=scratch,
        compiler_params=pltpu.CompilerParams(needs_layout_passes=False,
                                             use_tc_tiling_on_sc=False))
    def k(hscg, hsgc, stb, scg, dcg, sgc, dgc,
          aggg, aggc,
          r0_v, r1_v, r2_v, t0_v, t1_v, t2_v,
          s0_v, s1_v, s2_v, d0_v, d1_v, d2_v, i0_v, i1_v, i2_v,
          exb_v, agg_s,
          gs0, gs1, gs2, is0, is1, is2, ss0, ss1, ss2):
        rows = [r0_v, r1_v, r2_v]
        sdst = [t0_v, t1_v, t2_v]
        srcv = [s0_v, s1_v, s2_v]
        dstv = [d0_v, d1_v, d2_v]
        sciv = [i0_v, i1_v, i2_v]
        gsem = [gs0, gs1, gs2]
        isem = [is0, is1, is2]
        ssem = [ss0, ss1, ss2]
        rows_v = rows[0]
        cid = lax.axis_index("c")
        sid = lax.axis_index("s")

        lane = lax.iota(jnp.int32, 16)
        den_off = jnp.where(lane < 4, lane * K, 0)
        den_msk = jnp.where(lane < 4, 1.0, 0.0).astype(_f32)

        def run(hs, roff, srcE, dstE, aggo):
            def zrow(i, _):
                rows_v[i // (CE // 16), pl.ds((i % (CE // 16)) * 16, 16)] = (
                    jnp.zeros((16,), _f32))
                return 0
            lax.fori_loop(0, K * (CE // 16), zrow, 0)

            r0 = sid * RW
            def zcp(j, _):
                pltpu.sync_copy(rows_v, agg_s.at[pl.ds(r0 + j * K, K)])
                return 0
            lax.fori_loop(0, RW // K, zcp, 0)
            rem = RW - (RW // K) * K
            if rem:
                pltpu.sync_copy(rows_v.at[pl.ds(0, rem)],
                                agg_s.at[pl.ds(r0 + RW - rem, rem)])

            @pl.when(sid == NSUB - 1)
            def _():
                pltpu.sync_copy(rows_v.at[pl.ds(0, RREM)],
                                agg_s.at[pl.ds(RW * NSUB, RREM)])
            plsc.subcore_barrier()

            def compute(rv, tv, dv, cv):
                def lgrp(g, _):
                    e0 = g * 16
                    ids = jnp.full((16,), e0, jnp.int32) + lane
                    for h in range(H):
                        av = plsc.load_gather(
                            rv, [ids, jnp.full((16,), C + h, jnp.int32)])
                        bv = plsc.load_gather(
                            tv, [ids, jnp.full((16,), roff + h, jnp.int32)])
                        lv = av + bv
                        lv = jnp.where(lv >= 0.0, lv, 0.2 * lv)
                        idx4 = jnp.full((16,), e0 * 4 + h, jnp.int32) + lane * 4
                        plsc.store_scatter(exb_v, [idx4], jnp.exp(lv))
                    return 0
                lax.fori_loop(0, K // 16, lgrp, 0)
                for t in range(K // 16):
                    cv[t, :] = dv[pl.ds(t * 16, 16)]

                def escale(eb, _):
                    ex16 = exb_v[pl.ds(eb * 16, 16)]
                    e0 = eb * 4
                    for j in range(4):
                        e = e0 + j
                        dvv = jnp.take_along_axis(
                            ex16, jnp.where(lane < 4, lane + 4 * j, 0),
                            axis=0) * den_msk
                        rv[e, pl.ds(C, 16)] = dvv
                        for h in range(H):
                            sp = jnp.take_along_axis(
                                ex16, jnp.full((16,), 4 * j + h, jnp.int32),
                                axis=0)
                            for q in range(2):
                                o = h * DH + q * 16
                                rv[e, pl.ds(o, 16)] = rv[e, pl.ds(o, 16)] * sp
                    return 0
                lax.fori_loop(0, K // 4, escale, 0)

            def do_chunk(ch, b):
                bn = (b + 1) % NB
                bn2 = (b + 2) % NB

                @pl.when((ch >= 1) & (ch + 1 < NCH))
                def _():
                    pltpu.make_async_copy(
                        srcE.at[pl.ds(0, K)], srcv[bn], isem[bn]).wait()
                    pltpu.make_async_copy(
                        dstE.at[pl.ds(0, K)], dstv[bn], isem[bn]).wait()

                @pl.when((ch >= 2) & (ch + 1 < NCH))
                def _():
                    for q in range(K // 16):
                        pltpu.make_async_copy(
                            rows[bn].at[pl.ds(q * 16, 16)],
                            agg_s.at[sciv[bn].at[q]], ssem[bn]).wait()

                @pl.when(ch + 1 < NCH)
                def _():
                    for q in range(K // 16):
                        sl = pl.ds(q * 16, 16)
                        pltpu.async_copy(hs.at[srcv[bn].at[sl]],
                                         rows[bn].at[sl], gsem[bn])
                        pltpu.async_copy(stb.at[dstv[bn].at[sl]],
                                         sdst[bn].at[sl], gsem[bn])

                @pl.when(ch + 2 < NCH)
                def _():
                    base2 = sid * EW + (ch + 2) * K
                    pltpu.async_copy(srcE.at[pl.ds(base2, K)], srcv[bn2],
                                     isem[bn2])
                    pltpu.async_copy(dstE.at[pl.ds(base2, K)], dstv[bn2],
                                     isem[bn2])

                for q in range(K // 16):
                    sl = pl.ds(q * 16, 16)
                    pltpu.make_async_copy(hs.at[srcv[b].at[sl]],
                                          rows[b].at[sl], gsem[b]).wait()
                    pltpu.make_async_copy(stb.at[dstv[b].at[sl]],
                                          sdst[b].at[sl], gsem[b]).wait()

                compute(rows[b], sdst[b], dstv[b], sciv[b])

                for q in range(K // 16):
                    pltpu.async_copy(rows[b].at[pl.ds(q * 16, 16)],
                                     agg_s.at[sciv[b].at[q]], ssem[b],
                                     add=True)

            base0 = sid * EW
            pltpu.sync_copy(srcE.at[pl.ds(base0, K)], srcv[0])
            pltpu.sync_copy(dstE.at[pl.ds(base0, K)], dstv[0])
            pltpu.sync_copy(srcE.at[pl.ds(base0 + K, K)], srcv[1])
            pltpu.sync_copy(dstE.at[pl.ds(base0 + K, K)], dstv[1])
            for q in range(K // 16):
                sl = pl.ds(q * 16, 16)
                pltpu.async_copy(hs.at[srcv[0].at[sl]], rows[0].at[sl],
                                 gsem[0])
                pltpu.async_copy(stb.at[dstv[0].at[sl]], sdst[0].at[sl],
                                 gsem[0])

            def loop_body(ch, _):
                r = lax.rem(ch, NB)
                for b in range(NB):
                    @pl.when(r == b)
                    def _(b=b):
                        do_chunk(ch, b)
                return 0
            lax.fori_loop(0, NCH, loop_body, 0)

            for j in (NCH - 3, NCH - 2, NCH - 1):
                bj = j % NB
                for q in range(K // 16):
                    pltpu.make_async_copy(
                        rows[bj].at[pl.ds(q * 16, 16)],
                        agg_s.at[sciv[bj].at[q]], ssem[bj]).wait()
            plsc.subcore_barrier()

            pltpu.sync_copy(agg_s.at[pl.ds(r0, RW)], aggo.at[pl.ds(r0, RW)])

            @pl.when(sid == NSUB - 1)
            def _():
                pltpu.sync_copy(agg_s.at[pl.ds(RW * NSUB, RREM)],
                                aggo.at[pl.ds(RW * NSUB, RREM)])

        @pl.when(cid == 0)
        def _():
            run(hscg, 0, scg, dcg, aggg)

        @pl.when(cid == 1)
        def _():
            run(hsgc, 8, sgc, dgc, aggc)

    return k(hse_cg, hse_gc, sdst, src_cg, dst_cg, src_gc, dst_gc)


def kernel(x_chemical, x_gene, edge_index_cg, edge_index_gc,
           Wsrc, bsrc, Wdst, bdst, attn, Wout, bout):
    xc, xg = x_chemical, x_gene
    src_cg, dst_cg = edge_index_cg[0], edge_index_cg[1]
    src_gc, dst_gc = edge_index_gc[0], edge_index_gc[1]

    eye4 = jnp.eye(H, dtype=_f32)
    Ex = jnp.concatenate(
        [jnp.repeat(eye4, DH, axis=1), jnp.zeros((12, C), _f32)], axis=0)
    z12 = jnp.zeros((C, 12), _f32)
    z4 = jnp.zeros((C, 4), _f32)
    zb4 = jnp.zeros((4,), _f32)
    zb12 = jnp.zeros((12,), _f32)

    for l in range(L):
        A0 = (attn[l, 0][:, :, None] * eye4[:, None, :]).reshape(C, H)
        A1 = (attn[l, 1][:, :, None] * eye4[:, None, :]).reshape(C, H)
        Wts0, bts0 = Wsrc[l, 0] @ A0, bsrc[l, 0] @ A0
        Wtd0, btd0 = Wdst[l, 0] @ A0, bdst[l, 0] @ A0
        Wts1, bts1 = Wsrc[l, 1] @ A1, bsrc[l, 1] @ A1
        Wtd1, btd1 = Wdst[l, 1] @ A1, bdst[l, 1] @ A1
        Ms0 = jnp.concatenate([Wts0, z12], axis=1)
        Ms1 = jnp.concatenate([Wts1, z12], axis=1)
        Md = jnp.concatenate([
            jnp.concatenate([Wtd0, z12], axis=1),
            jnp.concatenate([z4, z4, Wtd1, z4], axis=1),
        ], axis=0)
        bs0 = jnp.concatenate([bts0, zb12])[None]
        bs1 = jnp.concatenate([bts1, zb12])[None]
        bd = jnp.concatenate([btd0, zb4, btd1, zb4])[None]

        hse_cg, hse_gc, sdst = _proj_call(
            xc, xg, Wsrc[l, 0], bsrc[l, 0][None], Wsrc[l, 1], bsrc[l, 1][None],
            Ms0, Ms1, Md, bs0, bs1, bd)

        aggg, aggc = _sc_edge(
            hse_cg, hse_gc, sdst, src_cg, dst_cg, src_gc, dst_gc)

        xg, xc = _out_call(aggg, aggc,
                           Wout[l, 1], bout[l, 1][None],
                           Wout[l, 0], bout[l, 0][None], xg, xc, Ex)

    return jnp.concatenate([xc, xg], axis=0)


# speedup vs baseline: 2.2089x; 1.0000x over previous
import functools

import jax
import jax.numpy as jnp
from jax import lax
from jax.experimental import pallas as pl
from jax.experimental.pallas import tpu as pltpu
from jax.experimental.pallas import tpu_sc as plsc

N = 10000
E = 320000
C = 128
H = 4
DH = 32
L = 2

NSUB = 16
EW = E // NSUB
K = 80
NCH = EW // K
RW = 624
RREM = N - RW * NSUB
RB = 1000
CE = C + 16

_f32 = jnp.float32


def _proj_body(xc, xg, Wcg, bcg, Wgc, bgc, Ms0, Ms1, Md, bs0, bs1, bd,
               hse_cg, hse_gc, sdst):
    xcb = xc[...]
    xgb = xg[...]
    hse_cg[:, 0:C] = jnp.dot(xcb, Wcg[...], preferred_element_type=_f32) + bcg[...]
    hse_cg[:, C:CE] = (jnp.dot(xcb, Ms0[...], preferred_element_type=_f32)
                       + bs0[...])
    hse_gc[:, 0:C] = jnp.dot(xgb, Wgc[...], preferred_element_type=_f32) + bgc[...]
    hse_gc[:, C:CE] = (jnp.dot(xgb, Ms1[...], preferred_element_type=_f32)
                       + bs1[...])
    sdst[...] = (jnp.dot(xgb, Md[...][0:C, :], preferred_element_type=_f32)
                 + jnp.dot(xcb, Md[...][C:2 * C, :], preferred_element_type=_f32)
                 + bd[...])


def _proj_call(xc, xg, Wcg, bcg, Wgc, bgc, Ms0, Ms1, Md, bs0, bs1, bd):
    row = lambda i: (i, 0)
    full = lambda i: (0, 0)
    return pl.pallas_call(
        _proj_body,
        grid=(N // RB,),
        in_specs=[
            pl.BlockSpec((RB, C), row), pl.BlockSpec((RB, C), row),
            pl.BlockSpec((C, C), full), pl.BlockSpec((1, C), full),
            pl.BlockSpec((C, C), full), pl.BlockSpec((1, C), full),
            pl.BlockSpec((C, 16), full), pl.BlockSpec((C, 16), full),
            pl.BlockSpec((2 * C, 16), full),
            pl.BlockSpec((1, 16), full), pl.BlockSpec((1, 16), full),
            pl.BlockSpec((1, 16), full),
        ],
        out_specs=[pl.BlockSpec((RB, CE), row), pl.BlockSpec((RB, CE), row),
                   pl.BlockSpec((RB, 16), row)],
        out_shape=[jax.ShapeDtypeStruct((N, CE), _f32),
                   jax.ShapeDtypeStruct((N, CE), _f32),
                   jax.ShapeDtypeStruct((N, 16), _f32)],
    )(xc, xg, Wcg, bcg, Wgc, bgc, Ms0, Ms1, Md, bs0, bs1, bd)


def _out_body(aggg, aggc, Wg, bg, Wc, bc, xg, xc, Ex, yg, yc):
    ex = Ex[...]
    eg = aggg[...]
    sg = jnp.dot(1.0 / (eg[:, C:CE] + 1e-16), ex, preferred_element_type=_f32)
    ag = eg[:, 0:C] * sg
    yg[...] = jnp.maximum(
        jnp.dot(ag, Wg[...], preferred_element_type=_f32) + bg[...], 0.0) + xg[...]
    ec = aggc[...]
    sc = jnp.dot(1.0 / (ec[:, C:CE] + 1e-16), ex, preferred_element_type=_f32)
    ac = ec[:, 0:C] * sc
    yc[...] = jnp.maximum(
        jnp.dot(ac, Wc[...], preferred_element_type=_f32) + bc[...], 0.0) + xc[...]


def _out_call(aggg, aggc, Wg, bg, Wc, bc, xg, xc, Ex):
    row = lambda i: (i, 0)
    full = lambda i: (0, 0)
    return pl.pallas_call(
        _out_body,
        grid=(N // RB,),
        in_specs=[
            pl.BlockSpec((RB, CE), row), pl.BlockSpec((RB, CE), row),
            pl.BlockSpec((C, C), full), pl.BlockSpec((1, C), full),
            pl.BlockSpec((C, C), full), pl.BlockSpec((1, C), full),
            pl.BlockSpec((RB, C), row), pl.BlockSpec((RB, C), row),
            pl.BlockSpec((16, C), full),
        ],
        out_specs=[pl.BlockSpec((RB, C), row), pl.BlockSpec((RB, C), row)],
        out_shape=[jax.ShapeDtypeStruct((N, C), _f32),
                   jax.ShapeDtypeStruct((N, C), _f32)],
    )(aggg, aggc, Wg, bg, Wc, bc, xg, xc, Ex)


def _sc_edge(hse_cg, hse_gc, sdst, src_cg, dst_cg, src_gc, dst_gc):
    mesh = plsc.VectorSubcoreMesh(core_axis_name="c", subcore_axis_name="s")
    out_type = [jax.ShapeDtypeStruct((N, CE), _f32),
                jax.ShapeDtypeStruct((N, CE), _f32)]
    NB = 3
    scratch = (
        [pltpu.VMEM((K, CE), _f32)] * NB
        + [pltpu.VMEM((K, 16), _f32)] * NB
        + [pltpu.VMEM((K,), jnp.int32)] * NB
        + [pltpu.VMEM((K,), jnp.int32)] * NB
        + [pltpu.VMEM((K // 16, 16), jnp.int32)] * NB
        + [pltpu.VMEM((4 * K,), _f32)]
        + [pltpu.VMEM_SHARED((N, CE), _f32)]
        + [pltpu.SemaphoreType.DMA] * (3 * NB)
    )

    @functools.partial(
        pl.kernel, mesh=mesh, out_type=out_type, scratch_types=scratch,
        compiler_params=pltpu.CompilerParams(needs_layout_passes=False,
                                             use_tc_tiling_on_sc=False))
    def k(hscg, hsgc, stb, scg, dcg, sgc, dgc,
          aggg, aggc,
          r0_v, r1_v, r2_v, t0_v, t1_v, t2_v,
          s0_v, s1_v, s2_v, d0_v, d1_v, d2_v, i0_v, i1_v, i2_v,
          exb_v, agg_s,
          gs0, gs1, gs2, is0, is1, is2, ss0, ss1, ss2):
        rows = [r0_v, r1_v, r2_v]
        sdst = [t0_v, t1_v, t2_v]
        srcv = [s0_v, s1_v, s2_v]
        dstv = [d0_v, d1_v, d2_v]
        sciv = [i0_v, i1_v, i2_v]
        gsem = [gs0, gs1, gs2]
        isem = [is0, is1, is2]
        ssem = [ss0, ss1, ss2]
        rows_v = rows[0]
        cid = lax.axis_index("c")
        sid = lax.axis_index("s")

        lane = lax.iota(jnp.int32, 16)
        den_msk = jnp.where(lane < 4, 1.0, 0.0).astype(_f32)

        def run(hs, roff, srcE, dstE, aggo):
            def zrow(i, _):
                rows_v[i // (CE // 16), pl.ds((i % (CE // 16)) * 16, 16)] = (
                    jnp.zeros((16,), _f32))
                return 0
            lax.fori_loop(0, K * (CE // 16), zrow, 0)

            r0 = sid * RW
            def zcp(j, _):
                pltpu.sync_copy(rows_v, agg_s.at[pl.ds(r0 + j * K, K)])
                return 0
            lax.fori_loop(0, RW // K, zcp, 0)
            rem = RW - (RW // K) * K
            if rem:
                pltpu.sync_copy(rows_v.at[pl.ds(0, rem)],
                                agg_s.at[pl.ds(r0 + RW - rem, rem)])

            @pl.when(sid == NSUB - 1)
            def _():
                pltpu.sync_copy(rows_v.at[pl.ds(0, RREM)],
                                agg_s.at[pl.ds(RW * NSUB, RREM)])
            plsc.subcore_barrier()

            def compute(rv, tv, dv, cv):
                def lgrp(g, _):
                    e0 = g * 16
                    ids = jnp.full((16,), e0, jnp.int32) + lane
                    for h in range(H):
                        av = plsc.load_gather(
                            rv, [ids, jnp.full((16,), C + h, jnp.int32)])
                        bv = plsc.load_gather(
                            tv, [ids, jnp.full((16,), roff + h, jnp.int32)])
                        lv = av + bv
                        lv = jnp.where(lv >= 0.0, lv, 0.2 * lv)
                        idx4 = jnp.full((16,), e0 * 4 + h, jnp.int32) + lane * 4
                        plsc.store_scatter(exb_v, [idx4], jnp.exp(lv))
                    return 0
                lax.fori_loop(0, K // 16, lgrp, 0)
                for t in range(K // 16):
                    cv[t, :] = dv[pl.ds(t * 16, 16)]

                def escale(eb, _):
                    ex16 = exb_v[pl.ds(eb * 16, 16)]
                    e0 = eb * 4
                    for j in range(4):
                        e = e0 + j
                        dvv = jnp.take_along_axis(
                            ex16, jnp.where(lane < 4, lane + 4 * j, 0),
                            axis=0) * den_msk
                        rv[e, pl.ds(C, 16)] = dvv
                        for h in range(H):
                            sp = jnp.take_along_axis(
                                ex16, jnp.full((16,), 4 * j + h, jnp.int32),
                                axis=0)
                            for q in range(2):
                                o = h * DH + q * 16
                                rv[e, pl.ds(o, 16)] = rv[e, pl.ds(o, 16)] * sp
                    return 0
                lax.fori_loop(0, K // 4, escale, 0)

            def do_chunk(ch, b):
                bn = (b + 1) % NB
                bn2 = (b + 2) % NB

                @pl.when((ch >= 1) & (ch + 1 < NCH))
                def _():
                    pltpu.make_async_copy(
                        srcE.at[pl.ds(0, K)], srcv[bn], isem[bn]).wait()
                    pltpu.make_async_copy(
                        dstE.at[pl.ds(0, K)], dstv[bn], isem[bn]).wait()

                @pl.when((ch >= 2) & (ch + 1 < NCH))
                def _():
                    for q in range(K // 16):
                        pltpu.make_async_copy(
                            rows[bn].at[pl.ds(q * 16, 16)],
                            agg_s.at[sciv[bn].at[q]], ssem[bn]).wait()

                @pl.when(ch + 1 < NCH)
                def _():
                    for q in range(K // 16):
                        sl = pl.ds(q * 16, 16)
                        pltpu.async_copy(hs.at[srcv[bn].at[sl]],
                                         rows[bn].at[sl], gsem[bn])
                        pltpu.async_copy(stb.at[dstv[bn].at[sl]],
                                         sdst[bn].at[sl], gsem[bn])

                @pl.when(ch + 2 < NCH)
                def _():
                    base2 = sid * EW + (ch + 2) * K
                    pltpu.async_copy(srcE.at[pl.ds(base2, K)], srcv[bn2],
                                     isem[bn2])
                    pltpu.async_copy(dstE.at[pl.ds(base2, K)], dstv[bn2],
                                     isem[bn2])

                for q in range(K // 16):
                    sl = pl.ds(q * 16, 16)
                    pltpu.make_async_copy(hs.at[srcv[b].at[sl]],
                                          rows[b].at[sl], gsem[b]).wait()
                    pltpu.make_async_copy(stb.at[dstv[b].at[sl]],
                                          sdst[b].at[sl], gsem[b]).wait()

                compute(rows[b], sdst[b], dstv[b], sciv[b])

                for q in range(K // 16):
                    pltpu.async_copy(rows[b].at[pl.ds(q * 16, 16)],
                                     agg_s.at[sciv[b].at[q]], ssem[b],
                                     add=True)

            base0 = sid * EW
            pltpu.sync_copy(srcE.at[pl.ds(base0, K)], srcv[0])
            pltpu.sync_copy(dstE.at[pl.ds(base0, K)], dstv[0])
            pltpu.sync_copy(srcE.at[pl.ds(base0 + K, K)], srcv[1])
            pltpu.sync_copy(dstE.at[pl.ds(base0 + K, K)], dstv[1])
            for q in range(K // 16):
                sl = pl.ds(q * 16, 16)
                pltpu.async_copy(hs.at[srcv[0].at[sl]], rows[0].at[sl],
                                 gsem[0])
                pltpu.async_copy(stb.at[dstv[0].at[sl]], sdst[0].at[sl],
                                 gsem[0])

            def loop_body(ch, _):
                r = lax.rem(ch, NB)
                for b in range(NB):
                    @pl.when(r == b)
                    def _(b=b):
                        do_chunk(ch, b)
                return 0
            lax.fori_loop(0, NCH, loop_body, 0)

            for j in (NCH - 3, NCH - 2, NCH - 1):
                bj = j % NB
                for q in range(K // 16):
                    pltpu.make_async_copy(
                        rows[bj].at[pl.ds(q * 16, 16)],
                        agg_s.at[sciv[bj].at[q]], ssem[bj]).wait()
            plsc.subcore_barrier()

            pltpu.sync_copy(agg_s.at[pl.ds(r0, RW)], aggo.at[pl.ds(r0, RW)])

            @pl.when(sid == NSUB - 1)
            def _():
                pltpu.sync_copy(agg_s.at[pl.ds(RW * NSUB, RREM)],
                                aggo.at[pl.ds(RW * NSUB, RREM)])

        @pl.when(cid == 0)
        def _():
            run(hscg, 0, scg, dcg, aggg)

        @pl.when(cid == 1)
        def _():
            run(hsgc, 8, sgc, dgc, aggc)

    return k(hse_cg, hse_gc, sdst, src_cg, dst_cg, src_gc, dst_gc)


def kernel(x_chemical, x_gene, edge_index_cg, edge_index_gc,
           Wsrc, bsrc, Wdst, bdst, attn, Wout, bout):
    xc, xg = x_chemical, x_gene
    src_cg, dst_cg = edge_index_cg[0], edge_index_cg[1]
    src_gc, dst_gc = edge_index_gc[0], edge_index_gc[1]

    eye4 = jnp.eye(H, dtype=_f32)
    Ex = jnp.concatenate(
        [jnp.repeat(eye4, DH, axis=1), jnp.zeros((12, C), _f32)], axis=0)
    z12 = jnp.zeros((C, 12), _f32)
    z4 = jnp.zeros((C, 4), _f32)
    zb4 = jnp.zeros((4,), _f32)
    zb12 = jnp.zeros((12,), _f32)

    for l in range(L):
        A0 = (attn[l, 0][:, :, None] * eye4[:, None, :]).reshape(C, H)
        A1 = (attn[l, 1][:, :, None] * eye4[:, None, :]).reshape(C, H)
        Wts0, bts0 = Wsrc[l, 0] @ A0, bsrc[l, 0] @ A0
        Wtd0, btd0 = Wdst[l, 0] @ A0, bdst[l, 0] @ A0
        Wts1, bts1 = Wsrc[l, 1] @ A1, bsrc[l, 1] @ A1
        Wtd1, btd1 = Wdst[l, 1] @ A1, bdst[l, 1] @ A1
        Ms0 = jnp.concatenate([Wts0, z12], axis=1)
        Ms1 = jnp.concatenate([Wts1, z12], axis=1)
        Md = jnp.concatenate([
            jnp.concatenate([Wtd0, z12], axis=1),
            jnp.concatenate([z4, z4, Wtd1, z4], axis=1),
        ], axis=0)
        bs0 = jnp.concatenate([bts0, zb12])[None]
        bs1 = jnp.concatenate([bts1, zb12])[None]
        bd = jnp.concatenate([btd0, zb4, btd1, zb4])[None]

        hse_cg, hse_gc, sdst = _proj_call(
            xc, xg, Wsrc[l, 0], bsrc[l, 0][None], Wsrc[l, 1], bsrc[l, 1][None],
            Ms0, Ms1, Md, bs0, bs1, bd)

        aggg, aggc = _sc_edge(
            hse_cg, hse_gc, sdst, src_cg, dst_cg, src_gc, dst_gc)

        xg, xc = _out_call(aggg, aggc,
                           Wout[l, 1], bout[l, 1][None],
                           Wout[l, 0], bout[l, 0][None], xg, xc, Ex)

    return jnp.concatenate([xc, xg], axis=0)
